# baseline (speedup 1.0000x reference)
import functools

import jax
import jax.numpy as jnp
from jax import lax
from jax.experimental import pallas as pl
from jax.experimental.pallas import tpu as pltpu
from jax.experimental.pallas import tpu_sc as plsc

N = 100000
NODE = 128
EDGE = 16
OUT = 128
HID = 128
DEG = 4
B = 1000
L = 100
E = N * DEG

NC = 2
NS = 16
NW = NC * NS
RPW = 3200
NPAD = NW * RPW
C = 32
NCHG = NPAD // C
CH0 = 144
CH1 = 56

T = 800
GRID = N // T
EPS = 1e-5

_mesh = plsc.VectorSubcoreMesh(
    core_axis_name="c", subcore_axis_name="s", num_cores=NC, num_subcores=NS)
_mesh1 = plsc.VectorSubcoreMesh(
    core_axis_name="c", subcore_axis_name="s", num_cores=1, num_subcores=NS)


def _slab(idx):
    idx = jnp.pad(idx, ((0, NPAD - N), (0, 0)))
    return idx.reshape(NCHG, C, DEG).transpose(0, 2, 1).reshape(
        NCHG, DEG * C)


def _make_gathersum(nsegs, twidths, tc_tiling=True, single=False):
    ntab = len(nsegs)
    chmax = NCHG // NS if single else CH0
    scratch = (
        [pltpu.VMEM((chmax, DEG * C), jnp.int32)] * ntab
        + [pltpu.VMEM((DEG * C, tw), jnp.float32)
           for tw in twidths for _ in range(2)]
        + [pltpu.VMEM((C, NODE), jnp.float32)] * (2 * ntab)
        + [pltpu.SemaphoreType.DMA] * 4
    )
    out_type = tuple(
        jax.ShapeDtypeStruct((NPAD, NODE), jnp.float32) for _ in range(ntab))
    if ntab == 1:
        out_type = out_type[0]

    @functools.partial(
        pl.kernel, out_type=out_type, mesh=_mesh1 if single else _mesh,
        scratch_types=scratch,
        compiler_params=pltpu.CompilerParams(use_tc_tiling_on_sc=tc_tiling))
    def k(*refs):
        tbl = [refs[2 * t] for t in range(ntab)]
        slab = [refs[2 * t + 1] for t in range(ntab)]
        out = list(refs[2 * ntab:3 * ntab])
        p = 3 * ntab
        idx_v = list(refs[p:p + ntab]); p += ntab
        rows = [[refs[p + 2 * t], refs[p + 2 * t + 1]] for t in range(ntab)]
        p += 2 * ntab
        acc = [[refs[p + 2 * t], refs[p + 2 * t + 1]] for t in range(ntab)]
        p += 2 * ntab
        gsem = [refs[p], refs[p + 1]]
        wsem = [refs[p + 2], refs[p + 3]]

        cid = lax.axis_index("c")
        sid = lax.axis_index("s")
        if single:
            gbase = sid * chmax
            npair = chmax // 2
        else:
            gbase = jnp.where(cid == 0, sid * CH0, NS * CH0 + sid * CH1)
            npair = jnp.where(cid == 0, CH0 // 2, CH1 // 2)
        base = gbase * C
        for t in range(ntab):
            if single:
                pltpu.sync_copy(slab[t].at[pl.ds(gbase, chmax)], idx_v[t])
            else:
                pltpu.sync_copy(slab[t].at[pl.ds(gbase, CH1)],
                                idx_v[t].at[pl.ds(0, CH1)])

                @pl.when(cid == 0)
                def _(_t=t):
                    pltpu.sync_copy(
                        slab[_t].at[pl.ds(gbase + CH1, CH0 - CH1)],
                        idx_v[_t].at[pl.ds(CH1, CH0 - CH1)])

            if nsegs[t] < NODE // 16:
                def zrow(r, carry, _t=t):
                    for b in range(2):
                        for s in range(NODE // 16):
                            acc[_t][b][r, pl.ds(s * 16, 16)] = jnp.zeros(
                                (16,), jnp.float32)
                    return carry
                lax.fori_loop(0, C, zrow, 0)

        def g_desc(t, j, b):
            return pltpu.make_async_copy(
                tbl[t].at[idx_v[t].at[j]], rows[t][b], gsem[b])

        def w_desc(t, j, b):
            return pltpu.make_async_copy(
                acc[t][b], out[t].at[pl.ds(base + j * C, C)], wsem[b])

        def sum_chunk(t, b):
            def srow(r, carry):
                rv = rows[t][b]
                for s in range(nsegs[t]):
                    sl = pl.ds(s * 16, 16)
                    acc[t][b][r, sl] = (rv[r, sl] + rv[C + r, sl]
                                        + rv[2 * C + r, sl] + rv[3 * C + r, sl])
                return carry

            lax.fori_loop(0, C, srow, 0)

        for t in range(ntab):
            g_desc(t, 0, 0).start()

        def pair(t2, carry):
            jA = 2 * t2
            jB = jA + 1
            for t in range(ntab):
                g_desc(t, jB, 1).start()

            @pl.when(t2 > 0)
            def _():
                for t in range(ntab):
                    w_desc(t, jA - 2, 0).wait()

            for t in range(ntab):
                g_desc(t, jA, 0).wait()
            for t in range(ntab):
                sum_chunk(t, 0)
            for t in range(ntab):
                w_desc(t, jA, 0).start()

            @pl.when(t2 < npair - 1)
            def _():
                for t in range(ntab):
                    g_desc(t, jA + 2, 0).start()

            @pl.when(t2 > 0)
            def _():
                for t in range(ntab):
                    w_desc(t, jB - 2, 1).wait()

            for t in range(ntab):
                g_desc(t, jB, 1).wait()
            for t in range(ntab):
                sum_chunk(t, 1)
            for t in range(ntab):
                w_desc(t, jB, 1).start()
            return carry

        lax.fori_loop(0, npair, pair, 0)
        for t in range(ntab):
            w_desc(t, 2 * npair - 2, 0).wait()
            w_desc(t, 2 * npair - 1, 1).wait()

    return k


def _gathersum_node_edge(table, idxn, bond, idxe):
    return _make_gathersum((NODE // 16, 1), (NODE, EDGE), tc_tiling=False)(
        table, idxn, bond, idxe)


def _gathersum_node(table, idxn):
    return _make_gathersum((NODE // 16,), (NODE,))(table, idxn)


def _softmax(z):
    m = jnp.max(z, axis=-1, keepdims=True)
    e = jnp.exp(z - m)
    return e / jnp.sum(e, axis=-1, keepdims=True)


def _dot(a, b):
    return jnp.dot(a, b, preferred_element_type=jnp.float32)


def _acc_stats(i, a, ssum_ref, ssq_ref):
    a3 = a.reshape(T // 8, 8, HID)
    ps = jnp.sum(a3, axis=0)
    pq = jnp.sum(a3 * a3, axis=0)

    @pl.when(i == 0)
    def _():
        ssum_ref[...] = jnp.zeros_like(ssum_ref)
        ssq_ref[...] = jnp.zeros_like(ssq_ref)

    ssum_ref[...] += ps
    ssq_ref[...] += pq


def _tcsm_body(x_ref, w_ref, b_ref, out_ref):
    out_ref[...] = _softmax(_dot(x_ref[...], w_ref[...]) + b_ref[...])


def _tcsm_add_body(x_ref, w_ref, b_ref, prev_ref, out_ref):
    out_ref[...] = prev_ref[...] + _softmax(
        _dot(x_ref[...], w_ref[...]) + b_ref[...])


def _tc3_body(x_ref, gn_ref, ge_ref, wself_ref, wdn_ref, wde_ref,
              act_ref, ssum_ref, ssq_ref):
    a = (_dot(x_ref[...], wself_ref[...]) + _dot(gn_ref[...], wdn_ref[...])
         + _dot(ge_ref[...], wde_ref[...]))
    act_ref[...] = a
    _acc_stats(pl.program_id(0), a, ssum_ref, ssq_ref)


def _bn_relu(act, ssum, ssq):
    mean = jnp.sum(ssum, axis=0, keepdims=True) * (1.0 / N)
    var = jnp.sum(ssq, axis=0, keepdims=True) * (1.0 / N) - mean * mean
    return jnp.maximum((act - mean) * lax.rsqrt(var + EPS), 0.0)


def _tcbn_body(act_ref, ssum_ref, ssq_ref, h_ref):
    h_ref[...] = _bn_relu(act_ref[...], ssum_ref[...], ssq_ref[...])


def _tc4_body(act_ref, ssum_ref, ssq_ref, aa_ref, w_ref, b_ref, fp_ref):
    h = _bn_relu(act_ref[...], ssum_ref[...], ssq_ref[...])
    fp_ref[...] = aa_ref[...] + _softmax(_dot(h, w_ref[...]) + b_ref[...])


def _row_spec(minor):
    return pl.BlockSpec((T, minor), lambda i: (i, 0))


def _fix_spec(shape):
    return pl.BlockSpec(shape, lambda i: (0, 0))


def kernel(atom, bond, node_neighbors, edge_neighbors,
           Wout0, bout0, Wout1, bout1, Wout2, bout2,
           Wself0, Wdeg0, Wself1, Wdeg1):
    idxn = _slab(node_neighbors)
    idxe = _slab(edge_neighbors)
    Wd0n, Wd0e = Wdeg0[:NODE], jnp.pad(Wdeg0[NODE:], ((0, NODE - EDGE), (0, 0)))
    Wd1n, Wd1e = Wdeg1[:HID], jnp.pad(Wdeg1[HID:], ((0, NODE - EDGE), (0, 0)))
    b0 = bout0.reshape(1, OUT)
    b1 = bout1.reshape(1, OUT)
    b2 = bout2.reshape(1, OUT)

    def tc_act(x, gn, ge_, wself, wdn, wde):
        return pl.pallas_call(
            _tc3_body,
            grid=(GRID,),
            in_specs=[
                _row_spec(HID), _row_spec(HID), _row_spec(NODE),
                _fix_spec((HID, HID)), _fix_spec((HID, HID)),
                _fix_spec((NODE, HID)),
            ],
            out_specs=[
                _row_spec(HID), _fix_spec((8, HID)), _fix_spec((8, HID)),
            ],
            out_shape=[
                jax.ShapeDtypeStruct((N, HID), jnp.float32),
                jax.ShapeDtypeStruct((8, HID), jnp.float32),
                jax.ShapeDtypeStruct((8, HID), jnp.float32),
            ],
        )(x, gn, ge_, wself, wdn, wde)

    aa = pl.pallas_call(
        _tcsm_body,
        grid=(GRID,),
        in_specs=[_row_spec(NODE), _fix_spec((NODE, OUT)),
                  _fix_spec((1, OUT))],
        out_specs=_row_spec(OUT),
        out_shape=jax.ShapeDtypeStruct((N, OUT), jnp.float32),
    )(atom, Wout0, b0)

    gn1, ge = _gathersum_node_edge(atom, idxn, bond, idxe)
    act1, s1, q1 = tc_act(atom, gn1, ge, Wself0, Wd0n, Wd0e)

    h1 = pl.pallas_call(
        _tcbn_body,
        grid=(GRID,),
        in_specs=[_row_spec(HID), _fix_spec((8, HID)), _fix_spec((8, HID))],
        out_specs=_row_spec(HID),
        out_shape=jax.ShapeDtypeStruct((N, HID), jnp.float32),
    )(act1, s1, q1)

    gn2 = _gathersum_node(h1, idxn)

    aa2 = pl.pallas_call(
        _tcsm_add_body,
        grid=(GRID,),
        in_specs=[_row_spec(HID), _fix_spec((HID, OUT)), _fix_spec((1, OUT)),
                  _row_spec(OUT)],
        out_specs=_row_spec(OUT),
        out_shape=jax.ShapeDtypeStruct((N, OUT), jnp.float32),
    )(h1, Wout1, b1, aa)

    act2, s2, q2 = tc_act(h1, gn2, ge, Wself1, Wd1n, Wd1e)

    fp = pl.pallas_call(
        _tc4_body,
        grid=(GRID,),
        in_specs=[
            _row_spec(HID), _fix_spec((8, HID)), _fix_spec((8, HID)),
            _row_spec(OUT), _fix_spec((HID, OUT)), _fix_spec((1, OUT)),
        ],
        out_specs=_row_spec(OUT),
        out_shape=jax.ShapeDtypeStruct((N, OUT), jnp.float32),
    )(act2, s2, q2, aa2, Wout2, b2)

    return fp.reshape(B, L, OUT)


# speedup vs baseline: 1.0326x; 1.0326x over previous
import functools

import jax
import jax.numpy as jnp
from jax import lax
from jax.experimental import pallas as pl
from jax.experimental.pallas import tpu as pltpu
from jax.experimental.pallas import tpu_sc as plsc

N = 100000
NODE = 128
EDGE = 16
OUT = 128
HID = 128
DEG = 4
B = 1000
L = 100
E = N * DEG

NC = 2
NS = 16
NW = NC * NS
RPW = 3200
NPAD = NW * RPW
C = 32
NCHG = NPAD // C
CH0 = 144
CH1 = 56

T = 800
GRID = N // T
EPS = 1e-5

_mesh = plsc.VectorSubcoreMesh(
    core_axis_name="c", subcore_axis_name="s", num_cores=NC, num_subcores=NS)
_mesh1 = plsc.VectorSubcoreMesh(
    core_axis_name="c", subcore_axis_name="s", num_cores=1, num_subcores=NS)


def _slab(idx):
    idx = jnp.pad(idx, ((0, NPAD - N), (0, 0)))
    return idx.reshape(NCHG, C, DEG).transpose(0, 2, 1).reshape(
        NCHG, DEG * C)


def _make_gathersum(nsegs, twidths, tc_tiling=True, single=False):
    ntab = len(nsegs)
    chmax = NCHG // NS if single else CH0
    scratch = (
        [pltpu.VMEM((chmax, DEG * C), jnp.int32)] * ntab
        + [pltpu.VMEM((DEG * C, tw), jnp.float32)
           for tw in twidths for _ in range(2)]
        + [pltpu.VMEM((C, NODE), jnp.float32)] * (2 * ntab)
        + [pltpu.SemaphoreType.DMA] * 4
    )
    out_type = tuple(
        jax.ShapeDtypeStruct((NPAD, NODE), jnp.float32) for _ in range(ntab))
    if ntab == 1:
        out_type = out_type[0]

    @functools.partial(
        pl.kernel, out_type=out_type, mesh=_mesh1 if single else _mesh,
        scratch_types=scratch,
        compiler_params=pltpu.CompilerParams(use_tc_tiling_on_sc=tc_tiling))
    def k(*refs):
        tbl = [refs[2 * t] for t in range(ntab)]
        slab = [refs[2 * t + 1] for t in range(ntab)]
        out = list(refs[2 * ntab:3 * ntab])
        p = 3 * ntab
        idx_v = list(refs[p:p + ntab]); p += ntab
        rows = [[refs[p + 2 * t], refs[p + 2 * t + 1]] for t in range(ntab)]
        p += 2 * ntab
        acc = [[refs[p + 2 * t], refs[p + 2 * t + 1]] for t in range(ntab)]
        p += 2 * ntab
        gsem = [refs[p], refs[p + 1]]
        wsem = [refs[p + 2], refs[p + 3]]

        cid = lax.axis_index("c")
        sid = lax.axis_index("s")
        if single:
            gbase = sid * chmax
            npair = chmax // 2
        else:
            gbase = jnp.where(cid == 0, sid * CH0, NS * CH0 + sid * CH1)
            npair = jnp.where(cid == 0, CH0 // 2, CH1 // 2)
        base = gbase * C
        for t in range(ntab):
            if single:
                pltpu.sync_copy(slab[t].at[pl.ds(gbase, chmax)], idx_v[t])
            else:
                pltpu.sync_copy(slab[t].at[pl.ds(gbase, CH1)],
                                idx_v[t].at[pl.ds(0, CH1)])

                @pl.when(cid == 0)
                def _(_t=t):
                    pltpu.sync_copy(
                        slab[_t].at[pl.ds(gbase + CH1, CH0 - CH1)],
                        idx_v[_t].at[pl.ds(CH1, CH0 - CH1)])

            if nsegs[t] < NODE // 16:
                def zrow(r, carry, _t=t):
                    for b in range(2):
                        for s in range(NODE // 16):
                            acc[_t][b][r, pl.ds(s * 16, 16)] = jnp.zeros(
                                (16,), jnp.float32)
                    return carry
                lax.fori_loop(0, C, zrow, 0)

        def g_desc(t, j, b):
            return pltpu.make_async_copy(
                tbl[t].at[idx_v[t].at[j]], rows[t][b], gsem[b])

        def w_desc(t, j, b):
            return pltpu.make_async_copy(
                acc[t][b], out[t].at[pl.ds(base + j * C, C)], wsem[b])

        def sum_chunk(t, b):
            def srow(r, carry):
                rv = rows[t][b]
                for s in range(nsegs[t]):
                    sl = pl.ds(s * 16, 16)
                    acc[t][b][r, sl] = (rv[r, sl] + rv[C + r, sl]
                                        + rv[2 * C + r, sl] + rv[3 * C + r, sl])
                return carry

            lax.fori_loop(0, C, srow, 0)

        for t in range(ntab):
            g_desc(t, 0, 0).start()

        def pair(t2, carry):
            jA = 2 * t2
            jB = jA + 1
            for t in range(ntab):
                g_desc(t, jB, 1).start()

            @pl.when(t2 > 0)
            def _():
                for t in range(ntab):
                    w_desc(t, jA - 2, 0).wait()

            for t in range(ntab):
                g_desc(t, jA, 0).wait()
            for t in range(ntab):
                sum_chunk(t, 0)
            for t in range(ntab):
                w_desc(t, jA, 0).start()

            @pl.when(t2 < npair - 1)
            def _():
                for t in range(ntab):
                    g_desc(t, jA + 2, 0).start()

            @pl.when(t2 > 0)
            def _():
                for t in range(ntab):
                    w_desc(t, jB - 2, 1).wait()

            for t in range(ntab):
                g_desc(t, jB, 1).wait()
            for t in range(ntab):
                sum_chunk(t, 1)
            for t in range(ntab):
                w_desc(t, jB, 1).start()
            return carry

        lax.fori_loop(0, npair, pair, 0)
        for t in range(ntab):
            w_desc(t, 2 * npair - 2, 0).wait()
            w_desc(t, 2 * npair - 1, 1).wait()

    return k


def _gathersum_node_edge(table, idxn, bond, idxe):
    return _make_gathersum((NODE // 16, 1), (NODE, EDGE), tc_tiling=False)(
        table, idxn, bond, idxe)


def _gathersum_node(table, idxn):
    return _make_gathersum((NODE // 16,), (NODE,))(table, idxn)


def _softmax(z):
    m = jnp.max(z, axis=-1, keepdims=True)
    e = jnp.exp(z - m)
    return e / jnp.sum(e, axis=-1, keepdims=True)


def _dot(a, b):
    return jnp.dot(a, b, preferred_element_type=jnp.float32)


def _acc_stats(i, a, ssum_ref, ssq_ref):
    a3 = a.reshape(T // 8, 8, HID)
    ps = jnp.sum(a3, axis=0)
    pq = jnp.sum(a3 * a3, axis=0)

    @pl.when(i == 0)
    def _():
        ssum_ref[...] = jnp.zeros_like(ssum_ref)
        ssq_ref[...] = jnp.zeros_like(ssq_ref)

    ssum_ref[...] += ps
    ssq_ref[...] += pq


def _tc1_body(x_ref, gn_ref, ge_ref, wout_ref, b_ref, wself_ref, wdn_ref,
              wde_ref, aa_ref, act_ref, ssum_ref, ssq_ref):
    x = x_ref[...]
    aa_ref[...] = _softmax(_dot(x, wout_ref[...]) + b_ref[...])
    a = (_dot(x, wself_ref[...]) + _dot(gn_ref[...], wdn_ref[...])
         + _dot(ge_ref[...], wde_ref[...]))
    act_ref[...] = a
    _acc_stats(pl.program_id(0), a, ssum_ref, ssq_ref)


def _tc3_body(x_ref, gn_ref, ge_ref, wself_ref, wdn_ref, wde_ref,
              act_ref, ssum_ref, ssq_ref):
    a = (_dot(x_ref[...], wself_ref[...]) + _dot(gn_ref[...], wdn_ref[...])
         + _dot(ge_ref[...], wde_ref[...]))
    act_ref[...] = a
    _acc_stats(pl.program_id(0), a, ssum_ref, ssq_ref)


def _bn_relu(act, ssum, ssq):
    mean = jnp.sum(ssum, axis=0, keepdims=True) * (1.0 / N)
    var = jnp.sum(ssq, axis=0, keepdims=True) * (1.0 / N) - mean * mean
    return jnp.maximum((act - mean) * lax.rsqrt(var + EPS), 0.0)


def _tc2_body(act_ref, ssum_ref, ssq_ref, aa_ref, w_ref, b_ref,
              h_ref, aa2_ref):
    h = _bn_relu(act_ref[...], ssum_ref[...], ssq_ref[...])
    h_ref[...] = h
    aa2_ref[...] = aa_ref[...] + _softmax(_dot(h, w_ref[...]) + b_ref[...])


def _tc4_body(act_ref, ssum_ref, ssq_ref, aa_ref, w_ref, b_ref, fp_ref):
    h = _bn_relu(act_ref[...], ssum_ref[...], ssq_ref[...])
    fp_ref[...] = aa_ref[...] + _softmax(_dot(h, w_ref[...]) + b_ref[...])


def _row_spec(minor):
    return pl.BlockSpec((T, minor), lambda i: (i, 0))


def _fix_spec(shape):
    return pl.BlockSpec(shape, lambda i: (0, 0))


def kernel(atom, bond, node_neighbors, edge_neighbors,
           Wout0, bout0, Wout1, bout1, Wout2, bout2,
           Wself0, Wdeg0, Wself1, Wdeg1):
    idxn = _slab(node_neighbors)
    idxe = _slab(edge_neighbors)
    Wd0n, Wd0e = Wdeg0[:NODE], jnp.pad(Wdeg0[NODE:], ((0, NODE - EDGE), (0, 0)))
    Wd1n, Wd1e = Wdeg1[:HID], jnp.pad(Wdeg1[HID:], ((0, NODE - EDGE), (0, 0)))
    b0 = bout0.reshape(1, OUT)
    b1 = bout1.reshape(1, OUT)
    b2 = bout2.reshape(1, OUT)

    def tc_act(x, gn, ge_, wself, wdn, wde):
        return pl.pallas_call(
            _tc3_body,
            grid=(GRID,),
            in_specs=[
                _row_spec(HID), _row_spec(HID), _row_spec(NODE),
                _fix_spec((HID, HID)), _fix_spec((HID, HID)),
                _fix_spec((NODE, HID)),
            ],
            out_specs=[
                _row_spec(HID), _fix_spec((8, HID)), _fix_spec((8, HID)),
            ],
            out_shape=[
                jax.ShapeDtypeStruct((N, HID), jnp.float32),
                jax.ShapeDtypeStruct((8, HID), jnp.float32),
                jax.ShapeDtypeStruct((8, HID), jnp.float32),
            ],
        )(x, gn, ge_, wself, wdn, wde)

    gn1, ge = _gathersum_node_edge(atom, idxn, bond, idxe)

    aa, act1, s1, q1 = pl.pallas_call(
        _tc1_body,
        grid=(GRID,),
        in_specs=[
            _row_spec(NODE), _row_spec(NODE), _row_spec(NODE),
            _fix_spec((NODE, OUT)), _fix_spec((1, OUT)),
            _fix_spec((NODE, HID)), _fix_spec((NODE, HID)),
            _fix_spec((NODE, HID)),
        ],
        out_specs=[
            _row_spec(OUT), _row_spec(HID),
            _fix_spec((8, HID)), _fix_spec((8, HID)),
        ],
        out_shape=[
            jax.ShapeDtypeStruct((N, OUT), jnp.float32),
            jax.ShapeDtypeStruct((N, HID), jnp.float32),
            jax.ShapeDtypeStruct((8, HID), jnp.float32),
            jax.ShapeDtypeStruct((8, HID), jnp.float32),
        ],
    )(atom, gn1, ge, Wout0, b0, Wself0, Wd0n, Wd0e)

    h1, aa2 = pl.pallas_call(
        _tc2_body,
        grid=(GRID,),
        in_specs=[
            _row_spec(HID), _fix_spec((8, HID)), _fix_spec((8, HID)),
            _row_spec(OUT), _fix_spec((HID, OUT)), _fix_spec((1, OUT)),
        ],
        out_specs=[_row_spec(HID), _row_spec(OUT)],
        out_shape=[
            jax.ShapeDtypeStruct((N, HID), jnp.float32),
            jax.ShapeDtypeStruct((N, OUT), jnp.float32),
        ],
    )(act1, s1, q1, aa, Wout1, b1)

    gn2 = _gathersum_node(h1, idxn)
    act2, s2, q2 = tc_act(h1, gn2, ge, Wself1, Wd1n, Wd1e)

    fp = pl.pallas_call(
        _tc4_body,
        grid=(GRID,),
        in_specs=[
            _row_spec(HID), _fix_spec((8, HID)), _fix_spec((8, HID)),
            _row_spec(OUT), _fix_spec((HID, OUT)), _fix_spec((1, OUT)),
        ],
        out_specs=_row_spec(OUT),
        out_shape=jax.ShapeDtypeStruct((N, OUT), jnp.float32),
    )(act2, s2, q2, aa2, Wout2, b2)

    return fp.reshape(B, L, OUT)
